# baseline (speedup 1.0000x reference)
import functools

import jax
import jax.numpy as jnp
from jax import lax
from jax.experimental import pallas as pl
from jax.experimental.pallas import tpu as pltpu
from jax.experimental.pallas import tpu_sc as plsc

_NC = 2
_NS = 16
_L = 16
_CHUNK = 128
_LANE = 128


@functools.cache
def _build(B: int, D: int):
    NW = _NC * _NS
    assert B % (NW * _CHUNK) == 0 and _LANE % D == 0
    b_per_w = B // NW
    n_chunks = b_per_w // _CHUNK
    pack = _LANE // D
    shift = pack.bit_length() - 1
    assert 1 << shift == pack
    groups = _CHUNK // _L
    mesh = plsc.VectorSubcoreMesh(core_axis_name="c", subcore_axis_name="s")

    @functools.partial(
        pl.kernel,
        out_type=jax.ShapeDtypeStruct((B,), jnp.float32),
        mesh=mesh,
        compiler_params=pltpu.CompilerParams(use_tc_tiling_on_sc=True,
                                             needs_layout_passes=False),
        scratch_types=[
            pltpu.VMEM((n_chunks, _CHUNK), jnp.int32),
            pltpu.VMEM((n_chunks, _CHUNK), jnp.int32),
            pltpu.VMEM((n_chunks, _CHUNK), jnp.int32),
            pltpu.VMEM((n_chunks, _CHUNK), jnp.int32),
            pltpu.VMEM((2, _CHUNK, _LANE), jnp.float32),
            pltpu.VMEM((2, _CHUNK, _LANE), jnp.float32),
            pltpu.VMEM((b_per_w,), jnp.float32),
            pltpu.SemaphoreType.DMA,
            pltpu.SemaphoreType.DMA,
        ],
    )
    def k(uidx_hbm, iidx_hbm, utab_hbm, itab_hbm, out_hbm,
          uidx_v, iidx_v, urid_v, irid_v, ubuf, ibuf, out_v, sem0, sem1):
        sems = (sem0, sem1)
        wid = lax.axis_index("s") * _NC + lax.axis_index("c")
        base = wid * b_per_w

        for j in range(n_chunks):
            pltpu.sync_copy(uidx_hbm.at[pl.ds(base + j * _CHUNK, _CHUNK)],
                            uidx_v.at[j])
            pltpu.sync_copy(iidx_hbm.at[pl.ds(base + j * _CHUNK, _CHUNK)],
                            iidx_v.at[j])
            for g in range(groups):
                sl = pl.ds(g * _L, _L)
                urid_v[j, sl] = lax.shift_right_logical(uidx_v[j, sl], shift)
                irid_v[j, sl] = lax.shift_right_logical(iidx_v[j, sl], shift)

        def issue(j):
            p = j % 2
            cu = pltpu.async_copy(utab_hbm.at[urid_v.at[j]],
                                  ubuf.at[p], sems[p])
            ci = pltpu.async_copy(itab_hbm.at[irid_v.at[j]],
                                  ibuf.at[p], sems[p])
            return cu, ci

        lanes = lax.iota(jnp.int32, _L)
        cols = [jnp.full((_L,), d, jnp.int32) for d in range(D)]
        sub_mask = jnp.full((_L,), pack - 1, jnp.int32)
        dval = jnp.full((_L,), D, jnp.int32)

        pending = issue(0)
        for j in range(n_chunks):
            nxt = issue(j + 1) if j + 1 < n_chunks else None
            pending[0].wait()
            pending[1].wait()
            pending = nxt
            pbuf = jnp.full((_L,), j % 2, jnp.int32)

            def body(g, carry, j=j, pbuf=pbuf):
                sl = pl.ds(g * _L, _L)
                row = jnp.full((_L,), g * _L, jnp.int32) + lanes
                ucol = (uidx_v[j, sl] & sub_mask) * dval
                icol = (iidx_v[j, sl] & sub_mask) * dval
                accs = [jnp.zeros((_L,), jnp.float32) for _ in range(4)]
                for d in range(D):
                    u = plsc.load_gather(ubuf, [pbuf, row, ucol + cols[d]])
                    v = plsc.load_gather(ibuf, [pbuf, row, icol + cols[d]])
                    accs[d % 4] = accs[d % 4] + u * v
                out_v[pl.ds(j * _CHUNK + g * _L, _L)] = (
                    (accs[0] + accs[1]) + (accs[2] + accs[3]))
                return carry

            lax.fori_loop(0, groups, body, 0)

        pltpu.sync_copy(out_v, out_hbm.at[pl.ds(base, b_per_w)])

    return k


def kernel(user_item_tuple, user_table, item_table):
    uid = user_item_tuple[:, 0].astype(jnp.int32)
    iid = user_item_tuple[:, 1].astype(jnp.int32)
    n, d = user_table.shape
    pack = _LANE // d
    one = lax.optimization_barrier(jnp.float32(1.0))
    utt = lax.optimization_barrier(user_table.T)
    itt = lax.optimization_barrier(item_table.T)
    ut = (utt.T * one).reshape(n // pack, _LANE)
    it = (itt.T * one).reshape(n // pack, _LANE)
    return _build(uid.shape[0], d)(uid, iid, ut, it)


# speedup vs baseline: 5.6585x; 5.6585x over previous
import functools

import jax
import jax.numpy as jnp
from jax import lax
from jax.experimental import pallas as pl
from jax.experimental.pallas import tpu as pltpu
from jax.experimental.pallas import tpu_sc as plsc

_NC = 2
_NS = 16
_L = 16
_K = 8


@functools.cache
def _build(B: int, D: int):
    NW = _NC * _NS
    assert B % (NW * _L) == 0 and D == 2 * _L
    b_per_w = B // NW
    nb = b_per_w // _K
    mesh = plsc.VectorSubcoreMesh(core_axis_name="c", subcore_axis_name="s")

    @functools.partial(
        pl.kernel,
        out_type=jax.ShapeDtypeStruct((B,), jnp.float32),
        mesh=mesh,
        compiler_params=pltpu.CompilerParams(use_tc_tiling_on_sc=True,
                                             needs_layout_passes=False),
        scratch_types=[
            pltpu.VMEM((b_per_w,), jnp.int32),
            pltpu.VMEM((b_per_w,), jnp.int32),
            pltpu.VMEM((_K, D, 128), jnp.float32),
            pltpu.VMEM((_K, D, 128), jnp.float32),
            pltpu.VMEM((b_per_w,), jnp.float32),
            pltpu.SemaphoreType.DMA,
        ],
    )
    def k(uidx_hbm, iidx_hbm, utt_hbm, itt_hbm, out_hbm,
          uidx_v, iidx_v, ublk, iblk, out_v, sem):
        wid = lax.axis_index("s") * _NC + lax.axis_index("c")
        base = wid * b_per_w
        pltpu.sync_copy(uidx_hbm.at[pl.ds(base, b_per_w)], uidx_v)
        pltpu.sync_copy(iidx_hbm.at[pl.ds(base, b_per_w)], iidx_v)

        lanes = lax.iota(jnp.int32, _L)
        hi = lanes + jnp.full((_L,), _L, jnp.int32)
        zero_i = jnp.zeros((_L,), jnp.int32)

        def batch_vecs(b):
            vu = uidx_v[pl.ds(lax.div(b, 2) * _L, _L)]
            vi = iidx_v[pl.ds(lax.div(b, 2) * _L, _L)]
            loff = lax.rem(b, 2) * _K
            return vu, vi, loff

        def extract(vec, lane_id):
            return jnp.sum(jnp.where(lanes == lane_id, vec, zero_i))

        def body(b, res):
            @pl.when(b >= 1)
            def _():
                for _ in range(2 * _K):
                    pltpu.make_async_copy(
                        utt_hbm.at[:, pl.ds(0, 128)], ublk.at[0], sem).wait()

            def compute(res):
                vu, vi, loff = batch_vecs(b - 1)
                for l in range(_K):
                    lane_id = jnp.full((_L,), loff + l, jnp.int32)
                    ru = extract(vu, lane_id)
                    ri = extract(vi, lane_id)
                    mu = jnp.full((_L,), ru & 127, jnp.int32)
                    mi = jnp.full((_L,), ri & 127, jnp.int32)
                    kf = jnp.full((_L,), l, jnp.int32)
                    u_lo = plsc.load_gather(ublk, [kf, lanes, mu])
                    u_hi = plsc.load_gather(ublk, [kf, hi, mu])
                    v_lo = plsc.load_gather(iblk, [kf, lanes, mi])
                    v_hi = plsc.load_gather(iblk, [kf, hi, mi])
                    dot = jnp.sum(u_lo * v_lo + u_hi * v_hi)
                    res = jnp.where(lanes == lane_id, dot, res)
                return res

            def flush(res):
                out_v[pl.ds(lax.div(b - 1, 2) * _L, _L)] = res
                return res

            res = lax.cond(b >= 1, compute, lambda r: r, res)
            res = lax.cond((b >= 1) & (lax.rem(b, 2) == 0), flush,
                           lambda r: r, res)

            @pl.when(b < nb)
            def _():
                vu, vi, loff = batch_vecs(b)
                rbu = lax.shift_right_logical(vu, 7)
                rbi = lax.shift_right_logical(vi, 7)
                for l in range(_K):
                    lane_id = jnp.full((_L,), loff + l, jnp.int32)
                    cu = pl.multiple_of(extract(rbu, lane_id) * 128, 128)
                    ci = pl.multiple_of(extract(rbi, lane_id) * 128, 128)
                    pltpu.async_copy(utt_hbm.at[:, pl.ds(cu, 128)],
                                     ublk.at[l], sem)
                    pltpu.async_copy(itt_hbm.at[:, pl.ds(ci, 128)],
                                     iblk.at[l], sem)
            return res

        lax.fori_loop(0, nb + 1, body, jnp.zeros((_L,), jnp.float32))
        pltpu.sync_copy(out_v, out_hbm.at[pl.ds(base, b_per_w)])

    return k


def kernel(user_item_tuple, user_table, item_table):
    uid = user_item_tuple[:, 0].astype(jnp.int32)
    iid = user_item_tuple[:, 1].astype(jnp.int32)
    n, d = user_table.shape
    return _build(uid.shape[0], d)(uid, iid, user_table.T, item_table.T)


# speedup vs baseline: 5.8282x; 1.0300x over previous
import functools

import jax
import jax.numpy as jnp
from jax import lax
from jax.experimental import pallas as pl
from jax.experimental.pallas import tpu as pltpu
from jax.experimental.pallas import tpu_sc as plsc

_NC = 2
_NS = 16
_L = 16
_K = 4


@functools.cache
def _build(B: int, D: int):
    NW = _NC * _NS
    assert B % (NW * _L) == 0 and D == 2 * _L
    b_per_w = B // NW
    nb = b_per_w // _K
    mesh = plsc.VectorSubcoreMesh(core_axis_name="c", subcore_axis_name="s")

    @functools.partial(
        pl.kernel,
        out_type=jax.ShapeDtypeStruct((B,), jnp.float32),
        mesh=mesh,
        compiler_params=pltpu.CompilerParams(use_tc_tiling_on_sc=True,
                                             needs_layout_passes=False),
        scratch_types=[
            pltpu.VMEM((b_per_w,), jnp.int32),
            pltpu.VMEM((b_per_w,), jnp.int32),
            pltpu.VMEM((2 * _K, D, 128), jnp.float32),
            pltpu.VMEM((2 * _K, D, 128), jnp.float32),
            pltpu.VMEM((b_per_w,), jnp.float32),
            pltpu.SemaphoreType.DMA,
            pltpu.SemaphoreType.DMA,
        ],
    )
    def k(uidx_hbm, iidx_hbm, utt_hbm, itt_hbm, out_hbm,
          uidx_v, iidx_v, ublk, iblk, out_v, sem0, sem1):
        sems = (sem0, sem1)
        wid = lax.axis_index("s") * _NC + lax.axis_index("c")
        base = wid * b_per_w
        pltpu.sync_copy(uidx_hbm.at[pl.ds(base, b_per_w)], uidx_v)
        pltpu.sync_copy(iidx_hbm.at[pl.ds(base, b_per_w)], iidx_v)

        lanes = lax.iota(jnp.int32, _L)
        hi = lanes + jnp.full((_L,), _L, jnp.int32)
        zero_i = jnp.zeros((_L,), jnp.int32)

        def batch_vecs(b):
            vu = uidx_v[pl.ds(lax.div(b, 4) * _L, _L)]
            vi = iidx_v[pl.ds(lax.div(b, 4) * _L, _L)]
            loff = lax.rem(b, 4) * _K
            return vu, vi, loff

        def extract(vec, lane_id):
            return jnp.sum(jnp.where(lanes == lane_id, vec, zero_i))

        def issue(b, parity):
            vu, vi, loff = batch_vecs(b)
            rbu = lax.shift_right_logical(vu, 7)
            rbi = lax.shift_right_logical(vi, 7)
            for l in range(_K):
                slot = parity * _K + l
                lane_id = jnp.full((_L,), loff + l, jnp.int32)
                cu = pl.multiple_of(extract(rbu, lane_id) * 128, 128)
                ci = pl.multiple_of(extract(rbi, lane_id) * 128, 128)
                pltpu.async_copy(utt_hbm.at[:, pl.ds(cu, 128)],
                                 ublk.at[slot], sems[parity])
                pltpu.async_copy(itt_hbm.at[:, pl.ds(ci, 128)],
                                 iblk.at[slot], sems[parity])

        def body(b, res):
            for parity in range(2):
                @pl.when((b < nb) & (lax.rem(b, 2) == parity))
                def _(parity=parity):
                    issue(b, parity)

            @pl.when(b >= 1)
            def _():
                for parity in range(2):
                    @pl.when(lax.rem(b - 1, 2) == parity)
                    def _(parity=parity):
                        for _ in range(2 * _K):
                            pltpu.make_async_copy(
                                utt_hbm.at[:, pl.ds(0, 128)], ublk.at[0],
                                sems[parity]).wait()

            def compute(res):
                vu, vi, loff = batch_vecs(b - 1)
                for l in range(_K):
                    lane_id = jnp.full((_L,), loff + l, jnp.int32)
                    ru = extract(vu, lane_id)
                    ri = extract(vi, lane_id)
                    mu = jnp.full((_L,), ru & 127, jnp.int32)
                    mi = jnp.full((_L,), ri & 127, jnp.int32)
                    pbase = lax.rem(b - 1, 2) * _K + l
                    kf = jnp.full((_L,), pbase, jnp.int32)
                    u_lo = plsc.load_gather(ublk, [kf, lanes, mu])
                    u_hi = plsc.load_gather(ublk, [kf, hi, mu])
                    v_lo = plsc.load_gather(iblk, [kf, lanes, mi])
                    v_hi = plsc.load_gather(iblk, [kf, hi, mi])
                    dot = jnp.sum(u_lo * v_lo + u_hi * v_hi)
                    res = jnp.where(lanes == lane_id, dot, res)
                return res

            def flush(res):
                out_v[pl.ds(lax.div(b - 1, 4) * _L, _L)] = res
                return res

            res = lax.cond(b >= 1, compute, lambda r: r, res)
            res = lax.cond((b >= 1) & (lax.rem(b, 4) == 0), flush,
                           lambda r: r, res)
            return res

        lax.fori_loop(0, nb + 1, body, jnp.zeros((_L,), jnp.float32))
        pltpu.sync_copy(out_v, out_hbm.at[pl.ds(base, b_per_w)])

    return k


def kernel(user_item_tuple, user_table, item_table):
    uid = user_item_tuple[:, 0].astype(jnp.int32)
    iid = user_item_tuple[:, 1].astype(jnp.int32)
    n, d = user_table.shape
    return _build(uid.shape[0], d)(uid, iid, user_table.T, item_table.T)
